# Initial kernel scaffold; baseline (speedup 1.0000x reference)
import jax
import jax.numpy as jnp
from jax.experimental import pallas as pl


def kernel(image1_descriptor, image2_descriptor, reliability, grid):
    raise NotImplementedError("write your pallas kernel here")


# speedup vs baseline: 2.9255x; 2.9255x over previous
import jax
import jax.numpy as jnp
from jax.experimental import pallas as pl

_B, _C, _H, _W = 2, 128, 32, 32
_HW = _H * _W
_NQ = 25
_A = float(_NQ - 1)
_K_COEF = 0.5


def _aploss_kernel(q_ref, imgf_ref, grid_ref, rel_ref, out_ref):
    gx = grid_ref[0, :, 0:1]
    gy = grid_ref[0, :, 1:2]
    x = (gx + 1.0) * (_W / 2.0) - 0.5
    y = (gy + 1.0) * (_H / 2.0) - 0.5
    x0 = jnp.floor(x)
    y0 = jnp.floor(y)
    x1 = x0 + 1.0
    y1 = y0 + 1.0
    wx1 = x - x0
    wx0 = 1.0 - wx1
    wy1 = y - y0
    wy0 = 1.0 - wy1

    iota_m = jax.lax.broadcasted_iota(jnp.int32, (_HW, _HW), 1)

    def tap_mat(xi, yi, wgt):
        valid = (xi >= 0.0) & (xi <= _W - 1.0) & (yi >= 0.0) & (yi <= _H - 1.0)
        w = jnp.where(valid, wgt, 0.0)
        xc = jnp.clip(xi, 0.0, _W - 1.0).astype(jnp.int32)
        yc = jnp.clip(yi, 0.0, _H - 1.0).astype(jnp.int32)
        idx = yc * _W + xc
        return jnp.where(iota_m == idx, w, 0.0)

    g_mat = (tap_mat(x0, y0, wx0 * wy0) + tap_mat(x1, y0, wx1 * wy0)
             + tap_mat(x0, y1, wx0 * wy1) + tap_mat(x1, y1, wx1 * wy1))

    db = jnp.dot(g_mat, imgf_ref[0], preferred_element_type=jnp.float32)
    scores = jnp.dot(q_ref[0], db.T, preferred_element_type=jnp.float32)
    t = jnp.clip(_A * (1.0 - scores), 0.0, _A)

    rn = jax.lax.broadcasted_iota(jnp.int32, (_HW, _HW), 0)
    label = ((jnp.abs(rn // _W - iota_m // _W) <= 4)
             & (jnp.abs(rn % _W - iota_m % _W) <= 4)).astype(jnp.float32)

    ap_acc = jnp.zeros((_HW, 1), jnp.float32)
    prev = jnp.zeros((_HW, 1), jnp.float32)
    for k in range(_NQ):
        s = jnp.clip((k + 1.0) - t, 0.0, 1.0)
        cumnbs = jnp.sum(s, axis=1, keepdims=True)
        cumrec = jnp.sum(s * label, axis=1, keepdims=True)
        ap_acc = ap_acc + cumrec * (cumrec - prev) / (1e-16 + cumnbs)
        prev = cumrec
    ap = ap_acc / prev

    relv = rel_ref[0]
    apq = 1.0 - (ap * relv + _K_COEF * (1.0 - relv))
    out_ref[0] = jnp.sum(apq, axis=0, keepdims=True)


def kernel(image1_descriptor, image2_descriptor, reliability, grid):
    q = image1_descriptor.reshape(_B, _C, _HW).transpose(0, 2, 1)
    imgf = image2_descriptor.reshape(_B, _C, _HW).transpose(0, 2, 1)
    gridf = grid.reshape(_B, _HW, 2)
    relf = reliability.reshape(_B, _HW, 1)

    partial = pl.pallas_call(
        _aploss_kernel,
        grid=(_B,),
        in_specs=[
            pl.BlockSpec((1, _HW, _C), lambda i: (i, 0, 0)),
            pl.BlockSpec((1, _HW, _C), lambda i: (i, 0, 0)),
            pl.BlockSpec((1, _HW, 2), lambda i: (i, 0, 0)),
            pl.BlockSpec((1, _HW, 1), lambda i: (i, 0, 0)),
        ],
        out_specs=pl.BlockSpec((1, 1, 1), lambda i: (i, 0, 0)),
        out_shape=jax.ShapeDtypeStruct((_B, 1, 1), jnp.float32),
    )(q, imgf, gridf, relf)
    return jnp.sum(partial) / float(_B * _HW)
